# baseline (speedup 1.0000x reference)
import functools

import jax
import jax.numpy as jnp
from jax import lax
from jax.experimental import pallas as pl
from jax.experimental.pallas import tpu as pltpu
from jax.experimental.pallas import tpu_sc as plsc


def _build(B, H, V, D, num_cores, num_subcores):
    NW = num_cores * num_subcores
    G = B // NW
    HS = 2
    n_blk = H // HS
    mesh = plsc.VectorSubcoreMesh(core_axis_name="c", subcore_axis_name="s")

    @functools.partial(
        pl.kernel,
        mesh=mesh,
        out_type=jax.ShapeDtypeStruct((H, D, B), jnp.float32),
        scratch_types=[
            pltpu.VMEM((HS * G,), jnp.int32),
            pltpu.VMEM((HS * G,), jnp.int32),
            pltpu.VMEM((HS * G, 2 * D), jnp.float32),
            pltpu.VMEM((HS * G, 2 * D), jnp.float32),
            pltpu.VMEM((HS, D, G), jnp.float32),
            pltpu.VMEM((HS, D, G), jnp.float32),
            pltpu.SemaphoreType.DMA((2,)),
            pltpu.SemaphoreType.DMA((2,)),
            pltpu.SemaphoreType.DMA((2,)),
        ],
        compiler_params=pltpu.CompilerParams(needs_layout_passes=False),
    )
    def run(idx_hbm, table_hbm, out_hbm, ix0, ix1, rows0, rows1,
            tr0, tr1, i_sem, g_sem, o_sem):
        wid = lax.axis_index("s") * num_cores + lax.axis_index("c")
        b0 = pl.multiple_of(wid * G, G)
        ix = (ix0, ix1)
        rows = (rows0, rows1)
        tr = (tr0, tr1)

        def i_copies(k, b):
            return [pltpu.make_async_copy(
                        idx_hbm.at[pl.ds((HS * k + h) * B + b0, G)],
                        ix[b].at[pl.ds(h * G, G)], i_sem.at[b])
                    for h in range(HS)]

        def g_copy(k, b):
            return pltpu.make_async_copy(
                table_hbm.at[ix[b]], rows[b], g_sem.at[b])

        def o_copy(k, b, h):
            return pltpu.make_async_copy(
                tr[b].at[h], out_hbm.at[HS * k + h, :, pl.ds(b0, G)],
                o_sem.at[b])

        row_ids = [lax.iota(jnp.int32, 16) + rb * 16
                   for rb in range(HS * G // 16)]

        def transpose_h(b, h):
            @plsc.parallel_loop(0, D, unroll=8)
            def _(d):
                cid = jnp.full((16,), 0, jnp.int32) + d
                for c in range(G // 16):
                    rb = h * (G // 16) + c
                    v = plsc.load_gather(rows[b], [row_ids[rb], cid])
                    tr[b][h, d, pl.ds(c * 16, 16)] = v

        for cp in i_copies(0, 0):
            cp.start()
        for cp in i_copies(1, 1):
            cp.start()
        for cp in i_copies(0, 0):
            cp.wait()
        g_copy(0, 0).start()

        def body(j, carry):
            for b in range(2):
                k = 2 * j + b
                nb = 1 - b
                @pl.when(k + 1 < n_blk)
                def _():
                    for cp in i_copies(k + 1, nb):
                        cp.wait()
                    g_copy(k + 1, nb).start()

                g_copy(k, b).wait()

                @pl.when(k + 2 < n_blk)
                def _():
                    for cp in i_copies(k + 2, b):
                        cp.start()

                @pl.when(k >= 2)
                def _():
                    for h in range(HS):
                        o_copy(k - 2, b, h).wait()

                for h in range(HS):
                    transpose_h(b, h)
                    o_copy(k, b, h).start()
            return carry

        lax.fori_loop(0, n_blk // 2, body, 0)
        for h in range(HS):
            o_copy(n_blk - 2, 0, h).wait()
            o_copy(n_blk - 1, 1, h).wait()

    return run


def kernel(x, table):
    B, H = x.shape
    V, D = table.shape
    info = plsc.get_sparse_core_info()
    run = _build(B, H, V, D, info.num_cores, info.num_subcores)
    table_p = jnp.pad(table, ((0, 0), (0, D)))
    idx = x.T.reshape(B * H).astype(jnp.int32)
    out_t = run(idx, table_p)
    return out_t.transpose(2, 0, 1)


# speedup vs baseline: 1.0006x; 1.0006x over previous
import functools

import jax
import jax.numpy as jnp
from jax import lax
from jax.experimental import pallas as pl
from jax.experimental.pallas import tpu as pltpu
from jax.experimental.pallas import tpu_sc as plsc


def _build(B, H, V, D, num_cores, num_subcores):
    NW = num_cores * num_subcores
    G = B // NW
    HS = 2
    n_blk = H // HS
    mesh = plsc.VectorSubcoreMesh(core_axis_name="c", subcore_axis_name="s")

    @functools.partial(
        pl.kernel,
        mesh=mesh,
        out_type=jax.ShapeDtypeStruct((H, D, B), jnp.float32),
        scratch_types=[
            pltpu.VMEM((HS * G,), jnp.int32),
            pltpu.VMEM((HS * G,), jnp.int32),
            pltpu.VMEM((HS * G, 2 * D), jnp.float32),
            pltpu.VMEM((HS * G, 2 * D), jnp.float32),
            pltpu.VMEM((HS, D, G), jnp.float32),
            pltpu.VMEM((HS, D, G), jnp.float32),
            pltpu.SemaphoreType.DMA((2,)),
            pltpu.SemaphoreType.DMA((2,)),
            pltpu.SemaphoreType.DMA((2,)),
        ],
        compiler_params=pltpu.CompilerParams(needs_layout_passes=False),
    )
    def run(idx_hbm, table_hbm, out_hbm, ix0, ix1, rows0, rows1,
            tr0, tr1, i_sem, g_sem, o_sem):
        wid = lax.axis_index("s") * num_cores + lax.axis_index("c")
        b0 = pl.multiple_of(wid * G, G)
        ix = (ix0, ix1)
        rows = (rows0, rows1)
        tr = (tr0, tr1)

        def i_copies(k, b):
            return [pltpu.make_async_copy(
                        idx_hbm.at[pl.ds((HS * k + h) * B + b0, G)],
                        ix[b].at[pl.ds(h * G, G)], i_sem.at[b])
                    for h in range(HS)]

        def g_copy(k, b):
            return pltpu.make_async_copy(
                table_hbm.at[ix[b]], rows[b], g_sem.at[b])

        def o_copy(k, b, h):
            return pltpu.make_async_copy(
                tr[b].at[h], out_hbm.at[HS * k + h, :, pl.ds(b0, G)],
                o_sem.at[b])

        row_ids = [lax.iota(jnp.int32, 16) + rb * 16
                   for rb in range(HS * G // 16)]

        def transpose_h(b, h):
            @plsc.parallel_loop(0, D, unroll=16)
            def _(d):
                cid = jnp.full((16,), 0, jnp.int32) + d
                for c in range(G // 16):
                    rb = h * (G // 16) + c
                    v = plsc.load_gather(rows[b], [row_ids[rb], cid])
                    tr[b][h, d, pl.ds(c * 16, 16)] = v

        for cp in i_copies(0, 0):
            cp.start()
        for cp in i_copies(1, 1):
            cp.start()
        for cp in i_copies(0, 0):
            cp.wait()
        g_copy(0, 0).start()

        def body(j, carry):
            for b in range(2):
                k = 2 * j + b
                nb = 1 - b
                @pl.when(k + 1 < n_blk)
                def _():
                    for cp in i_copies(k + 1, nb):
                        cp.wait()
                    g_copy(k + 1, nb).start()

                g_copy(k, b).wait()

                @pl.when(k + 2 < n_blk)
                def _():
                    for cp in i_copies(k + 2, b):
                        cp.start()

                @pl.when(k >= 2)
                def _():
                    for h in range(HS):
                        o_copy(k - 2, b, h).wait()

                for h in range(HS):
                    transpose_h(b, h)
                    o_copy(k, b, h).start()
            return carry

        lax.fori_loop(0, n_blk // 2, body, 0)
        for h in range(HS):
            o_copy(n_blk - 2, 0, h).wait()
            o_copy(n_blk - 1, 1, h).wait()

    return run


def kernel(x, table):
    B, H = x.shape
    V, D = table.shape
    info = plsc.get_sparse_core_info()
    run = _build(B, H, V, D, info.num_cores, info.num_subcores)
    table_p = jnp.pad(table, ((0, 0), (0, D)))
    idx = x.T.reshape(B * H).astype(jnp.int32)
    out_t = run(idx, table_p)
    return out_t.transpose(2, 0, 1)


# speedup vs baseline: 1.0025x; 1.0019x over previous
import functools

import jax
import jax.numpy as jnp
from jax import lax
from jax.experimental import pallas as pl
from jax.experimental.pallas import tpu as pltpu
from jax.experimental.pallas import tpu_sc as plsc


def _build(B, H, V, D, num_cores, num_subcores):
    NW = num_cores * num_subcores
    G = B // NW
    HS = 2
    n_blk = H // HS
    mesh = plsc.VectorSubcoreMesh(core_axis_name="c", subcore_axis_name="s")

    @functools.partial(
        pl.kernel,
        mesh=mesh,
        out_type=jax.ShapeDtypeStruct((H, D, B), jnp.float32),
        scratch_types=[
            pltpu.VMEM((H * G,), jnp.int32),
            pltpu.VMEM((HS * G, 2 * D), jnp.float32),
            pltpu.VMEM((HS * G, 2 * D), jnp.float32),
            pltpu.VMEM((HS, D, G), jnp.float32),
            pltpu.VMEM((HS, D, G), jnp.float32),
            pltpu.SemaphoreType.DMA((2,)),
            pltpu.SemaphoreType.DMA((2,)),
            pltpu.SemaphoreType.DMA((2,)),
        ],
        compiler_params=pltpu.CompilerParams(needs_layout_passes=False),
    )
    def run(idx_hbm, table_hbm, out_hbm, ix_all, rows0, rows1,
            tr0, tr1, i_sem, g_sem, o_sem):
        wid = lax.axis_index("s") * num_cores + lax.axis_index("c")
        b0 = pl.multiple_of(wid * G, G)
        rows = (rows0, rows1)
        tr = (tr0, tr1)

        pltpu.sync_copy(idx_hbm.at[pl.ds(wid * H * G, H * G)], ix_all)

        def g_copy(k, b):
            return pltpu.make_async_copy(
                table_hbm.at[ix_all.at[pl.ds(k * HS * G, HS * G)]],
                rows[b], g_sem.at[b])

        def o_copy(k, b, h):
            return pltpu.make_async_copy(
                tr[b].at[h], out_hbm.at[HS * k + h, :, pl.ds(b0, G)],
                o_sem.at[b])

        row_ids = [lax.iota(jnp.int32, 16) + rb * 16
                   for rb in range(HS * G // 16)]

        def transpose_h(b, h):
            @plsc.parallel_loop(0, D, unroll=8)
            def _(d):
                cid = jnp.full((16,), 0, jnp.int32) + d
                for c in range(G // 16):
                    rb = h * (G // 16) + c
                    v = plsc.load_gather(rows[b], [row_ids[rb], cid])
                    tr[b][h, d, pl.ds(c * 16, 16)] = v

        g_copy(0, 0).start()

        def body(j, carry):
            for b in range(2):
                k = 2 * j + b
                nb = 1 - b
                @pl.when(k + 1 < n_blk)
                def _():
                    g_copy(k + 1, nb).start()

                g_copy(k, b).wait()

                @pl.when(k >= 2)
                def _():
                    for h in range(HS):
                        o_copy(k - 2, b, h).wait()

                for h in range(HS):
                    transpose_h(b, h)
                    o_copy(k, b, h).start()
            return carry

        lax.fori_loop(0, n_blk // 2, body, 0)
        for h in range(HS):
            o_copy(n_blk - 2, 0, h).wait()
            o_copy(n_blk - 1, 1, h).wait()

    return run


def kernel(x, table):
    B, H = x.shape
    V, D = table.shape
    info = plsc.get_sparse_core_info()
    run = _build(B, H, V, D, info.num_cores, info.num_subcores)
    table_p = jnp.pad(table, ((0, 0), (0, D)))
    NW = info.num_cores * info.num_subcores
    G = B // NW
    idx = (x.T.astype(jnp.int32)
           .reshape(H, NW, G).transpose(1, 0, 2).reshape(B * H))
    out_t = run(idx, table_p)
    return out_t.transpose(2, 0, 1)
